# baseline (speedup 1.0000x reference)
import functools

import jax
import jax.numpy as jnp
from jax import lax
from jax.experimental import pallas as pl
from jax.experimental.pallas import tpu as pltpu
from jax.experimental.pallas import tpu_sc as plsc

TAU = 0.05
K = 10
NX = 1024
NY = 100000
D = 64
BLK = 2048
NB = (NY + BLK - 1) // BLK
NYP = NB * BLK
NGRP = NB * 128
NSG = NB * 8
SGL = 512

NEG = -1e30
BIGI = 1 << 30

NC = 2
NS = 16
NW = NC * NS
RPW = NX // NW
RB = 4
NBATCH = RPW // RB


def _normalize(x, eps=1e-12):
    n = jnp.sqrt(jnp.sum(x * x, axis=-1, keepdims=True))
    return x / jnp.maximum(n, eps)


def _tc_body(fx_ref, fy_ref, fxn_ref, fyn_ref, m3_ref, m2_ref):
    j = pl.program_id(0)

    fxn = _normalize(fx_ref[...])

    @pl.when(j == 0)
    def _init():
        fxn_ref[...] = fxn.astype(jnp.bfloat16).astype(jnp.float32)

    fybn = _normalize(fy_ref[...])
    rowg = j * BLK + lax.broadcasted_iota(jnp.int32, (BLK, D), 0)
    fybn = jnp.where(rowg < NY, fybn, 0.0)
    fyn_ref[...] = fybn.astype(jnp.bfloat16).astype(jnp.float32)

    sim = lax.dot_general(
        fxn.astype(jnp.bfloat16), fybn.astype(jnp.bfloat16),
        (((1,), (1,)), ((), ())),
        preferred_element_type=jnp.float32) * (1.0 / TAU)

    planes = [sim[:, s * 128:(s + 1) * 128] for s in range(16)]
    islast = j == NB - 1
    lane128 = lax.broadcasted_iota(jnp.int32, (NX, 128), 1)
    planes[13] = jnp.where(
        jnp.logical_and(islast, lane128 >= 32), NEG, planes[13])
    planes[14] = jnp.where(islast, NEG, planes[14])
    planes[15] = jnp.where(islast, NEG, planes[15])
    while len(planes) > 1:
        planes = [jnp.maximum(planes[2 * i], planes[2 * i + 1])
                  for i in range(len(planes) // 2)]
    M = planes[0]
    m3_ref[0] = M

    m2_ref[0] = jnp.max(M.reshape(NX, 8, 16), axis=2)


def _sel_body(m2_ref, sel_ref):
    S = m2_ref[...]
    ji = lax.broadcasted_iota(jnp.int32, (NB, NX, 8), 0)
    qi = lax.broadcasted_iota(jnp.int32, (NB, NX, 8), 2)
    I = ji * 8 + qi
    sels = []
    for _ in range(K):
        m = jnp.max(jnp.max(S, axis=0), axis=1, keepdims=True)
        cand = jnp.where(S == m[None], I, BIGI)
        am = jnp.min(jnp.min(cand, axis=0), axis=1, keepdims=True)
        S = jnp.where(I == am[None], NEG, S)
        sels.append(am)
    sel_ref[...] = jnp.concatenate(
        sels + [jnp.zeros((NX, 16 - K), jnp.int32)], axis=1)


@jax.jit
def _tc(fx, fy):
    return pl.pallas_call(
        _tc_body,
        grid=(NB,),
        in_specs=[
            pl.BlockSpec((NX, D), lambda j: (0, 0)),
            pl.BlockSpec((BLK, D), lambda j: (j, 0)),
        ],
        out_specs=[
            pl.BlockSpec((NX, D), lambda j: (0, 0)),
            pl.BlockSpec((BLK, D), lambda j: (j, 0)),
            pl.BlockSpec((1, NX, 128), lambda j: (j, 0, 0)),
            pl.BlockSpec((1, NX, 8), lambda j: (j, 0, 0)),
        ],
        out_shape=[
            jax.ShapeDtypeStruct((NX, D), jnp.float32),
            jax.ShapeDtypeStruct((NYP, D), jnp.float32),
            jax.ShapeDtypeStruct((NB, NX, 128), jnp.float32),
            jax.ShapeDtypeStruct((NB, NX, 8), jnp.float32),
        ],
    )(fx, fy)


@jax.jit
def _sel(m2):
    return pl.pallas_call(
        _sel_body,
        out_shape=jax.ShapeDtypeStruct((NX, 16), jnp.int32),
    )(m2)


@functools.lru_cache(maxsize=1)
def _get_mesh():
    return plsc.VectorSubcoreMesh(
        core_axis_name="c", subcore_axis_name="s",
        num_cores=NC, num_subcores=NS)


_LOG2E = 1.4426950408889634
_LN2 = 0.6931471805599453


def _exp_neg(x):
    y = x * _LOG2E
    n = (y - 0.5).astype(jnp.int32)
    f = y - n.astype(jnp.float32)
    u = f * _LN2
    p = 1.0 / 5040.0
    for c in (1.0 / 720.0, 1.0 / 120.0, 1.0 / 24.0, 1.0 / 6.0, 0.5, 1.0,
              1.0):
        p = p * u + c
    scale = plsc.bitcast((n + 127) << 23, jnp.float32)
    return p * scale


def _sort_desc(k, v):
    nk, sv = lax.sort_key_val(jnp.negative(k), v, dimension=0)
    return jnp.negative(nk), sv


def _merge_desc(dv, di, kk, vv):
    rv = lax.rev(kk, (0,))
    ri = lax.rev(vv, (0,))
    m = dv >= rv
    cv = jnp.where(m, dv, rv)
    ci = jnp.where(m, di, ri)
    return _sort_desc(cv, ci)


def _sc_body(fxn_hbm, fyn_hbm, m3_hbm, sel_hbm, vals_hbm, cols_hbm,
             selv, fxv, m3idx, m3v, gsel, gidx, gv, vout, cout, sem, semb):
    cid = lax.axis_index("c")
    sid = lax.axis_index("s")
    wid = sid * NC + cid
    r0 = wid * RPW
    iota = lax.iota(jnp.int32, 16)

    pltpu.sync_copy(sel_hbm.at[pl.ds(r0, RPW)], selv)
    pltpu.sync_copy(fxn_hbm.at[pl.ds(r0, RPW)], fxv)

    def _build_m3idx(r, carry):
        selrow = selv[r]
        idx = (selrow >> 3) * (NX * 8) + (r0 + r) * 8 + (selrow & 7)
        m3idx[pl.ds(r * 16, 16)] = idx
        return carry

    lax.fori_loop(0, RPW, _build_m3idx, 0)

    copies = []
    for c in range(4):
        copies.append(pltpu.async_copy(
            m3_hbm.at[m3idx.at[pl.ds(c * 128, 128)]],
            m3v.at[pl.ds(c * 128, 128)], sem))
    for cp in copies:
        cp.wait()

    def _select_groups(r, carry):
        selrow = selv[r]
        dv = jnp.full((16,), NEG, jnp.float32)
        di = jnp.zeros((16,), jnp.int32)
        for t in range(K):
            mv = m3v[r * 16 + t]
            gid = selrow[t] * 16 + iota
            kk, vv = _sort_desc(mv, gid)
            if t == 0:
                dv, di = kk, vv
            else:
                dv, di = _merge_desc(dv, di, kk, vv)
        gsel[r] = di
        for t in range(K):
            gg = di[t]
            rows = (gg >> 7) * BLK + (gg & 127) + 128 * iota
            gidx[pl.ds((r * K + t) * 16, 16)] = rows
        return carry

    lax.fori_loop(0, RPW, _select_groups, 0)

    GW = RB * K * 16

    def _fire(b, buf, psem):
        cps = []
        for c in range(GW // 80):
            cps.append(pltpu.async_copy(
                fyn_hbm.at[gidx.at[pl.ds(b * GW + c * 80, 80)]],
                gv.at[buf, pl.ds(c * 80, 80)], psem))
        return cps

    def _drain(b, buf, psem):
        for c in range(GW // 80):
            pltpu.make_async_copy(
                fyn_hbm.at[gidx.at[pl.ds(b * GW + c * 80, 80)]],
                gv.at[buf, pl.ds(c * 80, 80)], psem).wait()

    def _rescore(b, buf):
        for rl in range(RB):
            r = b * RB + rl
            di = gsel[r]
            zero = jnp.zeros((16,), jnp.float32)
            accs0 = tuple(zero for _ in range(K))

            def _dot(d, accs, _rl=rl, _r=r, _buf=buf):
                dvec = jnp.broadcast_to(d, (16,)).astype(jnp.int32)
                rvec = jnp.full((16,), _r, jnp.int32)
                bvec = jnp.full((16,), _buf, jnp.int32)
                fxs = plsc.load_gather(fxv, [rvec, dvec])
                new = []
                for t in range(K):
                    cv = plsc.load_gather(
                        gv, [bvec, (_rl * K + t) * 16 + iota, dvec])
                    new.append(accs[t] + fxs * cv)
                return tuple(new)

            accs = lax.fori_loop(0, D, _dot, accs0, unroll=8)

            fv = jnp.full((16,), NEG, jnp.float32)
            fc = jnp.zeros((16,), jnp.int32)
            for t in range(K):
                gg = di[t]
                rows = (gg >> 7) * BLK + (gg & 127) + 128 * iota
                sc = jnp.where(rows < NY, accs[t] * (1.0 / TAU), NEG)
                kk, vv = _sort_desc(sc, rows)
                if t == 0:
                    fv, fc = kk, vv
                else:
                    fv, fc = _merge_desc(fv, fc, kk, vv)

            e = _exp_neg(fv - fv[0])
            e = jnp.where(iota < K, e, 0.0)
            sv = jnp.broadcast_to(jnp.sum(e), (16,))
            rcp = jnp.ones((16,), jnp.float32) / sv
            rcp = rcp * (2.0 - sv * rcp)
            rcp = rcp * (2.0 - sv * rcp)
            p = e * rcp
            ck = jnp.where(iota < K, fc, BIGI)
            cs, ps = lax.sort_key_val(ck, p, dimension=0)
            vout[pl.ds(r * K, 16)] = ps
            cout[pl.ds(r * K, 16)] = cs

    _fire(0, 0, sem)

    def _pipe(b2, carry):
        be = 2 * b2
        _drain(be, 0, sem)
        _fire(be + 1, 1, semb)
        _rescore(be, 0)
        _drain(be + 1, 1, semb)

        @pl.when(be + 2 < NBATCH)
        def _():
            _fire(be + 2, 0, sem)

        _rescore(be + 1, 1)
        return carry

    lax.fori_loop(0, NBATCH // 2, _pipe, 0)

    pltpu.sync_copy(
        vout.at[pl.ds(0, RPW * K)], vals_hbm.at[pl.ds(r0 * K, RPW * K)])
    pltpu.sync_copy(
        cout.at[pl.ds(0, RPW * K)], cols_hbm.at[pl.ds(r0 * K, RPW * K)])


@functools.lru_cache(maxsize=1)
def _get_sc():
    return pl.kernel(
        _sc_body,
        out_type=[
            jax.ShapeDtypeStruct((NX * K,), jnp.float32),
            jax.ShapeDtypeStruct((NX * K,), jnp.int32),
        ],
        mesh=_get_mesh(),
        compiler_params=pltpu.CompilerParams(
            needs_layout_passes=False, use_tc_tiling_on_sc=False),
        scratch_types=[
            pltpu.VMEM((RPW, 16), jnp.int32),
            pltpu.VMEM((RPW, D), jnp.float32),
            pltpu.VMEM((RPW * 16,), jnp.int32),
            pltpu.VMEM((RPW * 16, 16), jnp.float32),
            pltpu.VMEM((RPW, 16), jnp.int32),
            pltpu.VMEM((RPW * K * 16,), jnp.int32),
            pltpu.VMEM((2, RB * K * 16, D), jnp.float32),
            pltpu.VMEM((RPW * K + 8,), jnp.float32),
            pltpu.VMEM((RPW * K + 8,), jnp.int32),
            pltpu.SemaphoreType.DMA,
            pltpu.SemaphoreType.DMA,
        ],
    )


def kernel(feat_x, feat_y):
    fx = feat_x[0]
    fy = feat_y[0]
    fxn, fyn, m3, m2 = _tc(fx, fy)
    sel = _sel(m2)
    m3f = m3.reshape(NB * NX * 8, 16)
    vals, cols = _get_sc()(fxn, fyn, m3f, sel)
    crow = jnp.arange(NX + 1, dtype=jnp.int32) * K
    return crow, cols, vals


# speedup vs baseline: 1.0642x; 1.0642x over previous
import functools

import jax
import jax.numpy as jnp
from jax import lax
from jax.experimental import pallas as pl
from jax.experimental.pallas import tpu as pltpu
from jax.experimental.pallas import tpu_sc as plsc

TAU = 0.05
K = 10
NX = 1024
NY = 100000
D = 64
BLK = 2048
NB = (NY + BLK - 1) // BLK
NYP = NB * BLK
NGRP = NB * 128
NSG = NB * 8
SGL = 512

NEG = -1e30
BIGI = 1 << 30

NC = 2
NS = 16
NW = NC * NS
RPW = NX // NW
RB = 2
NBATCH = RPW // RB


def _normalize(x, eps=1e-12):
    n = jnp.sqrt(jnp.sum(x * x, axis=-1, keepdims=True))
    return x / jnp.maximum(n, eps)


def _tc_body(fx_ref, fy_ref, fxn_ref, fyn_ref, m3_ref, m2_ref):
    j = pl.program_id(0)

    fxn = _normalize(fx_ref[...])

    @pl.when(j == 0)
    def _init():
        fxn_ref[...] = fxn.astype(jnp.bfloat16).astype(jnp.float32)

    fybn = _normalize(fy_ref[...])
    rowg = j * BLK + lax.broadcasted_iota(jnp.int32, (BLK, D), 0)
    fybn = jnp.where(rowg < NY, fybn, 0.0)
    fyn_ref[...] = fybn.astype(jnp.bfloat16).astype(jnp.float32)

    sim = lax.dot_general(
        fxn.astype(jnp.bfloat16), fybn.astype(jnp.bfloat16),
        (((1,), (1,)), ((), ())),
        preferred_element_type=jnp.float32) * (1.0 / TAU)

    planes = [sim[:, s * 128:(s + 1) * 128] for s in range(16)]
    islast = j == NB - 1
    lane128 = lax.broadcasted_iota(jnp.int32, (NX, 128), 1)
    planes[13] = jnp.where(
        jnp.logical_and(islast, lane128 >= 32), NEG, planes[13])
    planes[14] = jnp.where(islast, NEG, planes[14])
    planes[15] = jnp.where(islast, NEG, planes[15])
    while len(planes) > 1:
        planes = [jnp.maximum(planes[2 * i], planes[2 * i + 1])
                  for i in range(len(planes) // 2)]
    M = planes[0]
    m3_ref[0] = M

    m2_ref[0] = jnp.max(M.reshape(NX, 8, 16), axis=2)


def _sel_body(m2_ref, sel_ref):
    S = m2_ref[...]
    ji = lax.broadcasted_iota(jnp.int32, (NB, NX, 8), 0)
    qi = lax.broadcasted_iota(jnp.int32, (NB, NX, 8), 2)
    I = ji * 8 + qi
    sels = []
    for _ in range(K):
        m = jnp.max(jnp.max(S, axis=0), axis=1, keepdims=True)
        cand = jnp.where(S == m[None], I, BIGI)
        am = jnp.min(jnp.min(cand, axis=0), axis=1, keepdims=True)
        S = jnp.where(I == am[None], NEG, S)
        sels.append(am)
    sel_ref[...] = jnp.concatenate(
        sels + [jnp.zeros((NX, 16 - K), jnp.int32)], axis=1)


@jax.jit
def _tc(fx, fy):
    return pl.pallas_call(
        _tc_body,
        grid=(NB,),
        in_specs=[
            pl.BlockSpec((NX, D), lambda j: (0, 0)),
            pl.BlockSpec((BLK, D), lambda j: (j, 0)),
        ],
        out_specs=[
            pl.BlockSpec((NX, D), lambda j: (0, 0)),
            pl.BlockSpec((BLK, D), lambda j: (j, 0)),
            pl.BlockSpec((1, NX, 128), lambda j: (j, 0, 0)),
            pl.BlockSpec((1, NX, 8), lambda j: (j, 0, 0)),
        ],
        out_shape=[
            jax.ShapeDtypeStruct((NX, D), jnp.float32),
            jax.ShapeDtypeStruct((NYP, D), jnp.float32),
            jax.ShapeDtypeStruct((NB, NX, 128), jnp.float32),
            jax.ShapeDtypeStruct((NB, NX, 8), jnp.float32),
        ],
    )(fx, fy)


@jax.jit
def _sel(m2):
    return pl.pallas_call(
        _sel_body,
        out_shape=jax.ShapeDtypeStruct((NX, 16), jnp.int32),
    )(m2)


@functools.lru_cache(maxsize=1)
def _get_mesh():
    return plsc.VectorSubcoreMesh(
        core_axis_name="c", subcore_axis_name="s",
        num_cores=NC, num_subcores=NS)


_LOG2E = 1.4426950408889634
_LN2 = 0.6931471805599453


def _exp_neg(x):
    y = x * _LOG2E
    n = (y - 0.5).astype(jnp.int32)
    f = y - n.astype(jnp.float32)
    u = f * _LN2
    p = 1.0 / 5040.0
    for c in (1.0 / 720.0, 1.0 / 120.0, 1.0 / 24.0, 1.0 / 6.0, 0.5, 1.0,
              1.0):
        p = p * u + c
    scale = plsc.bitcast((n + 127) << 23, jnp.float32)
    return p * scale


def _sort_desc(k, v):
    nk, sv = lax.sort_key_val(jnp.negative(k), v, dimension=0)
    return jnp.negative(nk), sv


def _merge_desc(dv, di, kk, vv):
    rv = lax.rev(kk, (0,))
    ri = lax.rev(vv, (0,))
    m = dv >= rv
    cv = jnp.where(m, dv, rv)
    ci = jnp.where(m, di, ri)
    return _sort_desc(cv, ci)


def _sc_body(fxn_hbm, fyn_hbm, m3_hbm, sel_hbm, vals_hbm, cols_hbm,
             selv, fxv, m3idx, m3v, gsel, gidx, gv, vout, cout, sem, semb):
    cid = lax.axis_index("c")
    sid = lax.axis_index("s")
    wid = sid * NC + cid
    r0 = wid * RPW
    iota = lax.iota(jnp.int32, 16)

    pltpu.sync_copy(sel_hbm.at[pl.ds(r0, RPW)], selv)
    pltpu.sync_copy(fxn_hbm.at[pl.ds(r0, RPW)], fxv)

    def _build_m3idx(r, carry):
        selrow = selv[r]
        idx = (selrow >> 3) * (NX * 8) + (r0 + r) * 8 + (selrow & 7)
        m3idx[pl.ds(r * 16, 16)] = idx
        return carry

    lax.fori_loop(0, RPW, _build_m3idx, 0)

    copies = []
    for c in range(4):
        copies.append(pltpu.async_copy(
            m3_hbm.at[m3idx.at[pl.ds(c * 128, 128)]],
            m3v.at[pl.ds(c * 128, 128)], sem))
    for cp in copies:
        cp.wait()

    def _select_groups(r, carry):
        selrow = selv[r]
        dv = jnp.full((16,), NEG, jnp.float32)
        di = jnp.zeros((16,), jnp.int32)
        for t in range(K):
            mv = m3v[r * 16 + t]
            gid = selrow[t] * 16 + iota
            kk, vv = _sort_desc(mv, gid)
            if t == 0:
                dv, di = kk, vv
            else:
                dv, di = _merge_desc(dv, di, kk, vv)
        gsel[r] = di
        for t in range(K):
            gg = di[t]
            rows = (gg >> 7) * BLK + (gg & 127) + 128 * iota
            gidx[pl.ds((r * K + t) * 16, 16)] = rows
        return carry

    lax.fori_loop(0, RPW, _select_groups, 0)

    GW = RB * K * 16

    def _fire(b, buf, psem):
        cps = []
        for c in range(GW // 80):
            cps.append(pltpu.async_copy(
                fyn_hbm.at[gidx.at[pl.ds(b * GW + c * 80, 80)]],
                gv.at[buf, pl.ds(c * 80, 80)], psem))
        return cps

    def _drain(b, buf, psem):
        for c in range(GW // 80):
            pltpu.make_async_copy(
                fyn_hbm.at[gidx.at[pl.ds(b * GW + c * 80, 80)]],
                gv.at[buf, pl.ds(c * 80, 80)], psem).wait()

    def _rescore(b, buf):
        for rl in range(RB):
            r = b * RB + rl
            di = gsel[r]
            zero = jnp.zeros((16,), jnp.float32)
            accs0 = tuple(zero for _ in range(K))

            def _dot(d, accs, _rl=rl, _r=r, _buf=buf):
                dvec = jnp.broadcast_to(d, (16,)).astype(jnp.int32)
                rvec = jnp.full((16,), _r, jnp.int32)
                bvec = jnp.full((16,), _buf, jnp.int32)
                fxs = plsc.load_gather(fxv, [rvec, dvec])
                new = []
                for t in range(K):
                    cv = plsc.load_gather(
                        gv, [bvec, (_rl * K + t) * 16 + iota, dvec])
                    new.append(accs[t] + fxs * cv)
                return tuple(new)

            accs = lax.fori_loop(0, D, _dot, accs0, unroll=4)

            fv = jnp.full((16,), NEG, jnp.float32)
            fc = jnp.zeros((16,), jnp.int32)
            for t in range(K):
                gg = di[t]
                rows = (gg >> 7) * BLK + (gg & 127) + 128 * iota
                sc = jnp.where(rows < NY, accs[t] * (1.0 / TAU), NEG)
                kk, vv = _sort_desc(sc, rows)
                if t == 0:
                    fv, fc = kk, vv
                else:
                    fv, fc = _merge_desc(fv, fc, kk, vv)

            e = _exp_neg(fv - fv[0])
            e = jnp.where(iota < K, e, 0.0)
            sv = jnp.broadcast_to(jnp.sum(e), (16,))
            rcp = jnp.ones((16,), jnp.float32) / sv
            rcp = rcp * (2.0 - sv * rcp)
            rcp = rcp * (2.0 - sv * rcp)
            p = e * rcp
            ck = jnp.where(iota < K, fc, BIGI)
            cs, ps = lax.sort_key_val(ck, p, dimension=0)
            vout[pl.ds(r * K, 16)] = ps
            cout[pl.ds(r * K, 16)] = cs

    _fire(0, 0, sem)

    def _pipe(b2, carry):
        be = 2 * b2
        _drain(be, 0, sem)
        _fire(be + 1, 1, semb)
        _rescore(be, 0)
        _drain(be + 1, 1, semb)

        @pl.when(be + 2 < NBATCH)
        def _():
            _fire(be + 2, 0, sem)

        _rescore(be + 1, 1)
        return carry

    lax.fori_loop(0, NBATCH // 2, _pipe, 0)

    pltpu.sync_copy(
        vout.at[pl.ds(0, RPW * K)], vals_hbm.at[pl.ds(r0 * K, RPW * K)])
    pltpu.sync_copy(
        cout.at[pl.ds(0, RPW * K)], cols_hbm.at[pl.ds(r0 * K, RPW * K)])


@functools.lru_cache(maxsize=1)
def _get_sc():
    return pl.kernel(
        _sc_body,
        out_type=[
            jax.ShapeDtypeStruct((NX * K,), jnp.float32),
            jax.ShapeDtypeStruct((NX * K,), jnp.int32),
        ],
        mesh=_get_mesh(),
        compiler_params=pltpu.CompilerParams(
            needs_layout_passes=False, use_tc_tiling_on_sc=False),
        scratch_types=[
            pltpu.VMEM((RPW, 16), jnp.int32),
            pltpu.VMEM((RPW, D), jnp.float32),
            pltpu.VMEM((RPW * 16,), jnp.int32),
            pltpu.VMEM((RPW * 16, 16), jnp.float32),
            pltpu.VMEM((RPW, 16), jnp.int32),
            pltpu.VMEM((RPW * K * 16,), jnp.int32),
            pltpu.VMEM((2, RB * K * 16, D), jnp.float32),
            pltpu.VMEM((RPW * K + 8,), jnp.float32),
            pltpu.VMEM((RPW * K + 8,), jnp.int32),
            pltpu.SemaphoreType.DMA,
            pltpu.SemaphoreType.DMA,
        ],
    )


def kernel(feat_x, feat_y):
    fx = feat_x[0]
    fy = feat_y[0]
    fxn, fyn, m3, m2 = _tc(fx, fy)
    sel = _sel(m2)
    m3f = m3.reshape(NB * NX * 8, 16)
    vals, cols = _get_sc()(fxn, fyn, m3f, sel)
    crow = jnp.arange(NX + 1, dtype=jnp.int32) * K
    return crow, cols, vals
